# baseline (speedup 1.0000x reference)
import jax
import jax.numpy as jnp
from jax.experimental import pallas as pl
from jax.experimental.pallas import tpu as pltpu

_TOPK = 8


def _oemerge(a, b):
    if len(a) == 1:
        return [jnp.maximum(a[0], b[0]), jnp.minimum(a[0], b[0])]
    c = _oemerge(a[0::2], b[0::2])
    d = _oemerge(a[1::2], b[1::2])
    out = [c[0]]
    for i in range(len(d) - 1):
        out.append(jnp.maximum(d[i], c[i + 1]))
        out.append(jnp.minimum(d[i], c[i + 1]))
    out.append(d[-1])
    return out


def _top8_merge(a, b):
    m = [jnp.maximum(a[i], b[7 - i]) for i in range(8)]
    for dist in (4, 2, 1):
        nm = list(m)
        for i in range(8):
            if (i & dist) == 0:
                nm[i] = jnp.maximum(m[i], m[i + dist])
                nm[i + dist] = jnp.minimum(m[i], m[i + dist])
        m = nm
    return m


def _tc_body(xe_ref, xo_ref, main_ref, det_ref, do_s, mo_s):
    p = pl.program_id(2)

    @pl.when(p == 0)
    def _compute():
        xe = xe_ref[0]
        xo = xo_ref[0]
        low2 = (xe + xo) * 0.5
        high = xe - xo
        T2 = high.shape[0]

        cur = [[jnp.abs(high[i * 8:(i + 1) * 8, :])] for i in range(T2 // 8)]
        while len(cur) > 1:
            nxt = []
            for i in range(0, len(cur), 2):
                a, b = cur[i], cur[i + 1]
                if len(a) < 8:
                    nxt.append(_oemerge(a, b))
                else:
                    nxt.append(_top8_merge(a, b))
            cur = nxt
        S = jnp.concatenate(cur[0], axis=0)

        mx = None
        for _ in range(_TOPK):
            mx = jnp.max(S, axis=0, keepdims=True)
            S = jnp.where(S >= mx, jnp.float32(-1.0), S)
        thresh = mx

        det = jnp.where(jnp.abs(high) >= thresh, high * 0.5,
                        jnp.zeros_like(high))
        main_ref[0] = low2
        det_ref[0] = det
        do_s[...] = -det
        mo_s[...] = low2

    @pl.when(p == 1)
    def _write_odd():
        main_ref[0] = mo_s[...]
        det_ref[0] = do_s[...]


def kernel(x):
    B, T, F = x.shape
    T2 = T // 2
    FB = min(256, F)
    NF = F // FB
    xr = x.reshape(B, T2, 2 * F)

    spec_e = pl.BlockSpec((1, T2, FB), lambda b, fb, p: (b, 0, fb))
    spec_o = pl.BlockSpec((1, T2, FB), lambda b, fb, p: (b, 0, NF + fb))
    spec_out = pl.BlockSpec((1, T2, FB), lambda b, fb, p: (b, 0, p * NF + fb))

    main_r, det_r = pl.pallas_call(
        _tc_body,
        grid=(B, NF, 2),
        in_specs=[spec_e, spec_o],
        out_specs=[spec_out, spec_out],
        out_shape=[
            jax.ShapeDtypeStruct((B, T2, 2 * F), jnp.float32),
            jax.ShapeDtypeStruct((B, T2, 2 * F), jnp.float32),
        ],
        scratch_shapes=[
            pltpu.VMEM((T2, FB), jnp.float32),
            pltpu.VMEM((T2, FB), jnp.float32),
        ],
    )(xr, xr)
    return main_r.reshape(B, T, F), det_r.reshape(B, T, F)
